# Initial kernel scaffold; baseline (speedup 1.0000x reference)
import jax
import jax.numpy as jnp
from jax.experimental import pallas as pl


def kernel(feats):
    raise NotImplementedError("write your pallas kernel here")


# speedup vs baseline: 46.6971x; 46.6971x over previous
import functools

import jax
import jax.numpy as jnp
from jax import lax
from jax.experimental import pallas as pl
from jax.experimental.pallas import tpu as pltpu
from jax.experimental.pallas import tpu_sc as plsc

_N = 200000
_NV = 16
_NB = 128
_NC = 2
_NS = 16
_NW = _NC * _NS
_ROWS_W = _N // _NW
_CHUNK = 625
_NCHUNK = _ROWS_W // _CHUNK

_mesh = plsc.VectorSubcoreMesh(core_axis_name="c", subcore_axis_name="s")


def _wid():
    return lax.axis_index("s") * _NC + lax.axis_index("c")


@functools.partial(
    pl.kernel,
    out_type=jax.ShapeDtypeStruct((_NW * 2 * _NV,), jnp.float32),
    mesh=_mesh,
    scratch_types=[
        pltpu.VMEM((_ROWS_W * _NV,), jnp.float32),
        pltpu.VMEM((2 * _NV,), jnp.float32),
    ],
)
def _minmax_k(feats_hbm, out_hbm, buf, mm):
    wid = _wid()
    pltpu.sync_copy(feats_hbm.at[pl.ds(wid * _ROWS_W * _NV, _ROWS_W * _NV)], buf)

    def body(i, carry):
        mn, mx = carry
        x = buf[pl.ds(i * _NV, _NV)]
        return jnp.minimum(mn, x), jnp.maximum(mx, x)

    mn0 = buf[pl.ds(0, _NV)]
    mn, mx = lax.fori_loop(1, _ROWS_W, body, (mn0, mn0))
    mm[pl.ds(0, _NV)] = mn
    mm[pl.ds(_NV, _NV)] = mx
    pltpu.sync_copy(mm, out_hbm.at[pl.ds(wid * 2 * _NV, 2 * _NV)])


@functools.partial(
    pl.kernel,
    out_type=(
        jax.ShapeDtypeStruct((_N * _NV,), jnp.int32),
        jax.ShapeDtypeStruct((_N * _NV,), jnp.float32),
    ),
    mesh=_mesh,
    compiler_params=pltpu.CompilerParams(needs_layout_passes=False),
    scratch_types=[
        pltpu.VMEM((_NW * 2 * _NV,), jnp.float32),
        pltpu.VMEM((_NB * _NV,), jnp.float32),
        pltpu.VMEM((_CHUNK * _NV,), jnp.float32),
        pltpu.VMEM((_CHUNK * _NV,), jnp.int32),
        pltpu.VMEM((_CHUNK * _NV,), jnp.float32),
    ],
)
def _map_k(feats_hbm, mm_hbm, bins_hbm, regs_hbm, pbuf, linbuf, xbuf, bbuf, rbuf):
    wid = _wid()
    base = wid * _ROWS_W * _NV
    pltpu.sync_copy(mm_hbm, pbuf)

    def red(i, carry):
        mn, mx = carry
        return (
            jnp.minimum(mn, pbuf[pl.ds(i * 2 * _NV, _NV)]),
            jnp.maximum(mx, pbuf[pl.ds(i * 2 * _NV + _NV, _NV)]),
        )

    fmin, fmax = lax.fori_loop(
        1, _NW, red, (pbuf[pl.ds(0, _NV)], pbuf[pl.ds(_NV, _NV)])
    )

    inv_div = jnp.float32(1.0) / jnp.float32(_NB - 1)
    inv_div_v = jnp.broadcast_to(inv_div, (_NV,))

    def lrow(j, _):
        tj = jnp.broadcast_to(j, (_NV,)).astype(jnp.float32) * inv_div_v
        linbuf[pl.ds(j * _NV, _NV)] = fmin * (jnp.float32(1.0) - tj) + fmax * tj
        return 0

    lax.fori_loop(0, _NB, lrow, 0)

    t1 = jnp.float32(1.0) * inv_div
    lin1 = fmin * (jnp.float32(1.0) - t1) + fmax * t1
    step = lin1 - fmin
    inv_step = jnp.float32(1.0) / step
    inv_delta = jnp.float32(_NB - 1) / (fmax - fmin)
    lane = lax.iota(jnp.int32, _NV)
    c127 = jnp.broadcast_to(jnp.int32(_NB - 1), (_NV,))

    def chunk(ci, _):
        ebase = base + ci * _CHUNK * _NV
        pltpu.sync_copy(feats_hbm.at[pl.ds(ebase, _CHUNK * _NV)], xbuf)

        def row(r, _):
            x = xbuf[pl.ds(r * _NV, _NV)]
            q = (x - fmin) * inv_delta
            a = jnp.minimum(q.astype(jnp.int32), c127)
            fa = a * _NV + lane
            ga = plsc.load_gather(linbuf, [fa])
            gn = plsc.load_gather(
                linbuf, [jnp.minimum(a + 1, c127) * _NV + lane]
            )
            ca = (ga - x) * jnp.float32(1e6) <= jnp.float32(0.5)
            cn = ((gn - x) * jnp.float32(1e6) <= jnp.float32(0.5)) & (a < c127)
            b = a + ca.astype(jnp.int32) + cn.astype(jnp.int32) - 1
            gb = plsc.load_gather(linbuf, [b * _NV + lane])
            bbuf[pl.ds(r * _NV, _NV)] = b
            rbuf[pl.ds(r * _NV, _NV)] = jnp.maximum(x - gb, jnp.float32(0.0)) * inv_step
            return 0

        lax.fori_loop(0, _CHUNK, row, 0)
        pltpu.sync_copy(bbuf, bins_hbm.at[pl.ds(ebase, _CHUNK * _NV)])
        pltpu.sync_copy(rbuf, regs_hbm.at[pl.ds(ebase, _CHUNK * _NV)])
        return 0

    lax.fori_loop(0, _NCHUNK, chunk, 0)


def kernel(feats):
    partials = _minmax_k(feats.reshape(-1))
    bins, regs = _map_k(feats.reshape(-1), partials)
    return bins.reshape(_N, _NV), regs.reshape(_N, _NV)


# speedup vs baseline: 58.4763x; 1.2522x over previous
import functools

import jax
import jax.numpy as jnp
from jax import lax
from jax.experimental import pallas as pl
from jax.experimental.pallas import tpu as pltpu
from jax.experimental.pallas import tpu_sc as plsc

_N = 200000
_NV = 16
_NB = 128
_NC = 2
_NS = 16
_NW = _NC * _NS
_ROWS_W = _N // _NW
_CHUNK = 625
_NCHUNK = _ROWS_W // _CHUNK

_mesh = plsc.VectorSubcoreMesh(core_axis_name="c", subcore_axis_name="s")


def _wid():
    return lax.axis_index("s") * _NC + lax.axis_index("c")


@functools.partial(
    pl.kernel,
    out_type=jax.ShapeDtypeStruct((_NW * 2 * _NV,), jnp.float32),
    mesh=_mesh,
    scratch_types=[
        pltpu.VMEM((_ROWS_W * _NV,), jnp.float32),
        pltpu.VMEM((2 * _NV,), jnp.float32),
    ],
)
def _minmax_k(feats_hbm, out_hbm, buf, mm):
    wid = _wid()
    pltpu.sync_copy(feats_hbm.at[pl.ds(wid * _ROWS_W * _NV, _ROWS_W * _NV)], buf)

    def body(i, carry):
        mn, mx = carry
        x = buf[pl.ds(i * _NV, _NV)]
        return jnp.minimum(mn, x), jnp.maximum(mx, x)

    mn0 = buf[pl.ds(0, _NV)]
    mn, mx = lax.fori_loop(1, _ROWS_W, body, (mn0, mn0))
    mm[pl.ds(0, _NV)] = mn
    mm[pl.ds(_NV, _NV)] = mx
    pltpu.sync_copy(mm, out_hbm.at[pl.ds(wid * 2 * _NV, 2 * _NV)])


@functools.partial(
    pl.kernel,
    out_type=(
        jax.ShapeDtypeStruct((_N * _NV,), jnp.int32),
        jax.ShapeDtypeStruct((_N * _NV,), jnp.float32),
    ),
    mesh=_mesh,
    compiler_params=pltpu.CompilerParams(needs_layout_passes=False),
    scratch_types=[
        pltpu.VMEM((_NW * 2 * _NV,), jnp.float32),
        pltpu.VMEM((_NB * _NV,), jnp.float32),
        pltpu.VMEM((_CHUNK * _NV,), jnp.float32),
        pltpu.VMEM((_CHUNK * _NV,), jnp.int32),
        pltpu.VMEM((_CHUNK * _NV,), jnp.float32),
    ],
)
def _map_k(feats_hbm, mm_hbm, bins_hbm, regs_hbm, pbuf, linbuf, xbuf, bbuf, rbuf):
    wid = _wid()
    base = wid * _ROWS_W * _NV
    pltpu.sync_copy(mm_hbm, pbuf)

    def red(i, carry):
        mn, mx = carry
        return (
            jnp.minimum(mn, pbuf[pl.ds(i * 2 * _NV, _NV)]),
            jnp.maximum(mx, pbuf[pl.ds(i * 2 * _NV + _NV, _NV)]),
        )

    fmin, fmax = lax.fori_loop(
        1, _NW, red, (pbuf[pl.ds(0, _NV)], pbuf[pl.ds(_NV, _NV)])
    )

    inv_div = jnp.float32(1.0) / jnp.float32(_NB - 1)
    inv_div_v = jnp.broadcast_to(inv_div, (_NV,))

    def lrow(j, _):
        tj = jnp.broadcast_to(j, (_NV,)).astype(jnp.float32) * inv_div_v
        linbuf[pl.ds(j * _NV, _NV)] = fmin * (jnp.float32(1.0) - tj) + fmax * tj
        return 0

    lax.fori_loop(0, _NB, lrow, 0)

    t1 = jnp.float32(1.0) * inv_div
    lin1 = fmin * (jnp.float32(1.0) - t1) + fmax * t1
    step = lin1 - fmin
    inv_step = jnp.float32(1.0) / step
    inv_delta = jnp.float32(_NB - 1) / (fmax - fmin)
    lane = lax.iota(jnp.int32, _NV)
    c127 = jnp.broadcast_to(jnp.int32(_NB - 1), (_NV,))
    thr = jnp.float32(5e-7)

    def chunk(ci, _):
        ebase = base + ci * _CHUNK * _NV
        pltpu.sync_copy(feats_hbm.at[pl.ds(ebase, _CHUNK * _NV)], xbuf)

        @plsc.parallel_loop(0, _CHUNK, unroll=4)
        def row(r):
            x = xbuf[pl.ds(r * _NV, _NV)]
            q = (x - fmin) * inv_delta
            a = jnp.minimum(q.astype(jnp.int32), c127)
            fa = a * _NV + lane
            ga = plsc.load_gather(linbuf, [fa])
            gn = plsc.load_gather(
                linbuf, [jnp.minimum(a + 1, c127) * _NV + lane]
            )
            ca = ga - x <= thr
            cn = (gn - x <= thr) & (a < c127)
            b = a + ca.astype(jnp.int32) + cn.astype(jnp.int32) - 1
            gb = plsc.load_gather(linbuf, [b * _NV + lane])
            bbuf[pl.ds(r * _NV, _NV)] = b
            rbuf[pl.ds(r * _NV, _NV)] = jnp.maximum(x - gb, jnp.float32(0.0)) * inv_step

        pltpu.sync_copy(bbuf, bins_hbm.at[pl.ds(ebase, _CHUNK * _NV)])
        pltpu.sync_copy(rbuf, regs_hbm.at[pl.ds(ebase, _CHUNK * _NV)])
        return 0

    lax.fori_loop(0, _NCHUNK, chunk, 0)


def kernel(feats):
    partials = _minmax_k(feats.reshape(-1))
    bins, regs = _map_k(feats.reshape(-1), partials)
    return bins.reshape(_N, _NV), regs.reshape(_N, _NV)
